# baseline (speedup 1.0000x reference)
import functools

import jax
import jax.numpy as jnp
from jax import lax
from jax.experimental import pallas as pl
from jax.experimental.pallas import tpu as pltpu
from jax.experimental.pallas import tpu_sc as plsc

E = 800000
D = 64
NC = 2
NS = 16
NW = NC * NS
L = 16
C = 128
NB = 4
SUP = 1280
NSUB = SUP // C
NSUP = E // SUP
KMAX = (NSUP + NW - 1) // NW

_mesh = plsc.VectorSubcoreMesh(core_axis_name="c", subcore_axis_name="s")


@functools.partial(
    pl.kernel,
    out_type=jax.ShapeDtypeStruct((E, D), jnp.float32),
    mesh=_mesh,
    compiler_params=pltpu.CompilerParams(use_tc_tiling_on_sc=False),
    scratch_types=[
        pltpu.VMEM((SUP,), jnp.int32),
        pltpu.VMEM((SUP,), jnp.int32),
        pltpu.VMEM((SUP,), jnp.int32),
        pltpu.VMEM((SUP,), jnp.int32),
        pltpu.VMEM((NB, C, D), jnp.float32),
        pltpu.SemaphoreType.DMA((NB,)),
        pltpu.SemaphoreType.DMA((NB,)),
    ],
)
def _sc_lookup(a0_h, a1_h, a2_h, tab_h, out_h, a0_v, a1_v, a2_v, idx_v,
               rows_v, gsem, ssem):
    wid = lax.axis_index("s") * NC + lax.axis_index("c")

    def step(k, carry):
        sup = k * NW + wid

        @pl.when(sup < NSUP)
        def _():
            ebase = sup * SUP
            pltpu.sync_copy(a0_h.at[pl.ds(ebase, SUP)], a0_v)
            pltpu.sync_copy(a1_h.at[pl.ds(ebase, SUP)], a1_v)
            pltpu.sync_copy(a2_h.at[pl.ds(ebase, SUP)], a2_v)
            for g in range(SUP // L):
                s = pl.ds(g * L, L)
                idx_v[s] = a0_v[s] * 12 + a1_v[s] * 2 + a2_v[s]

            gd = [None] * NSUB
            sd = [None] * NSUB
            for j in range(NSUB):
                b = j % NB
                if j >= NB:
                    sd[j - NB].wait()
                gd[j] = pltpu.async_copy(
                    tab_h.at[idx_v.at[pl.ds(j * C, C)]], rows_v.at[b],
                    gsem.at[b])
                if j >= 1:
                    gd[j - 1].wait()
                    sd[j - 1] = pltpu.async_copy(
                        rows_v.at[(j - 1) % NB],
                        out_h.at[pl.ds(ebase + (j - 1) * C, C)],
                        ssem.at[(j - 1) % NB])
            gd[NSUB - 1].wait()
            sd[NSUB - 1] = pltpu.async_copy(
                rows_v.at[(NSUB - 1) % NB],
                out_h.at[pl.ds(ebase + (NSUB - 1) * C, C)],
                ssem.at[(NSUB - 1) % NB])
            for j in range(NSUB - NB, NSUB):
                sd[j].wait()

        return carry

    lax.fori_loop(0, KMAX, step, 0)


def kernel(edge_attr, W0, W1, W2):
    ea = edge_attr.astype(jnp.int32)
    a0 = ea[:, 0]
    a1 = ea[:, 1]
    a2 = ea[:, 2]
    tab = (W0[:, None, None, :] + W1[None, :, None, :]
           + W2[None, None, :, :]).reshape(60, D)
    return _sc_lookup(a0, a1, a2, tab)


# speedup vs baseline: 1.4703x; 1.4703x over previous
import functools

import jax
import jax.numpy as jnp
from jax import lax
from jax.experimental import pallas as pl
from jax.experimental.pallas import tpu as pltpu
from jax.experimental.pallas import tpu_sc as plsc

E = 800000
D = 64
NC = 2
NS = 16
NW = NC * NS
L = 16
C = 128
NB = 4
SUP = 1280
NSUB = SUP // C
NSUP = E // SUP
KMAX = (NSUP + NW - 1) // NW

_mesh = plsc.VectorSubcoreMesh(core_axis_name="c", subcore_axis_name="s")


@functools.partial(
    pl.kernel,
    out_type=jax.ShapeDtypeStruct((E, D), jnp.float32),
    mesh=_mesh,
    compiler_params=pltpu.CompilerParams(use_tc_tiling_on_sc=False,
                                         needs_layout_passes=False),
    scratch_types=[
        pltpu.VMEM_SHARED((60, D), jnp.float32),
        pltpu.VMEM((3 * SUP,), jnp.int32),
        pltpu.VMEM((SUP,), jnp.int32),
        pltpu.VMEM((NB, C, D), jnp.float32),
        pltpu.SemaphoreType.DMA((NB,)),
        pltpu.SemaphoreType.DMA((NB,)),
    ],
)
def _sc_lookup(attr_h, tab_h, out_h, tab_v, attr_v, idx_v, rows_v, gsem,
               ssem):
    wid = lax.axis_index("s") * NC + lax.axis_index("c")

    @pl.when(lax.axis_index("s") == 0)
    def _():
        pltpu.sync_copy(tab_h, tab_v)

    plsc.subcore_barrier()
    iota3 = lax.iota(jnp.int32, L) * 3

    def step(k, carry):
        sup = k * NW + wid

        @pl.when(sup < NSUP)
        def _():
            ebase = sup * SUP
            pltpu.sync_copy(attr_h.at[pl.ds(3 * ebase, 3 * SUP)], attr_v)
            for g in range(SUP // L):
                pos = iota3 + (3 * g * L)
                a0 = plsc.load_gather(attr_v, [pos])
                a1 = plsc.load_gather(attr_v, [pos + 1])
                a2 = plsc.load_gather(attr_v, [pos + 2])
                idx_v[pl.ds(g * L, L)] = a0 * 12 + a1 * 2 + a2

            gd = [None] * NSUB
            sd = [None] * NSUB
            for j in range(NSUB):
                b = j % NB
                if j >= NB:
                    sd[j - NB].wait()
                gd[j] = pltpu.async_copy(
                    tab_v.at[idx_v.at[pl.ds(j * C, C)]], rows_v.at[b],
                    gsem.at[b])
                if j >= 1:
                    gd[j - 1].wait()
                    sd[j - 1] = pltpu.async_copy(
                        rows_v.at[(j - 1) % NB],
                        out_h.at[pl.ds(ebase + (j - 1) * C, C)],
                        ssem.at[(j - 1) % NB])
            gd[NSUB - 1].wait()
            sd[NSUB - 1] = pltpu.async_copy(
                rows_v.at[(NSUB - 1) % NB],
                out_h.at[pl.ds(ebase + (NSUB - 1) * C, C)],
                ssem.at[(NSUB - 1) % NB])
            for j in range(NSUB - NB, NSUB):
                sd[j].wait()

        return carry

    lax.fori_loop(0, KMAX, step, 0)


def kernel(edge_attr, W0, W1, W2):
    attr_flat = edge_attr.astype(jnp.int32).reshape(3 * E)
    tab = (W0[:, None, None, :] + W1[None, :, None, :]
           + W2[None, None, :, :]).reshape(60, D)
    return _sc_lookup(attr_flat, tab)
